# Initial kernel scaffold; baseline (speedup 1.0000x reference)
import jax
import jax.numpy as jnp
from jax.experimental import pallas as pl


def kernel(x, node_features, adj_matrix, conv_w, conv_b):
    raise NotImplementedError("write your pallas kernel here")


# speedup vs baseline: 19.4830x; 19.4830x over previous
import functools

import jax
import jax.numpy as jnp
from jax import lax
from jax.experimental import pallas as pl
from jax.experimental.pallas import tpu as pltpu
from jax.experimental.pallas import tpu_sc as plsc

N_NODES = 10000
D_FEAT = 128
K_TOP = 8
NW = 32
ROWS_PER_W = 313
N_CHUNK = N_NODES // 16
GCAP = 64
IDXBUF = 10048

NEG_INF = float("-inf")


def _sc_topk(node_features, adj_matrix):
    mesh = plsc.VectorSubcoreMesh(core_axis_name="c", subcore_axis_name="s")

    @functools.partial(
        pl.kernel,
        out_type=jax.ShapeDtypeStruct((N_NODES, K_TOP * D_FEAT), jnp.float32),
        mesh=mesh,
        scratch_types=[
            pltpu.VMEM((N_NODES,), jnp.int32),
            pltpu.VMEM((IDXBUF,), jnp.int32),
            pltpu.VMEM((GCAP, D_FEAT), jnp.float32),
            pltpu.VMEM((K_TOP * D_FEAT,), jnp.float32),
            pltpu.SemaphoreType.DMA,
        ],
        compiler_params=pltpu.CompilerParams(needs_layout_passes=False),
    )
    def sc_kernel(f_hbm, adj_hbm, out_hbm, rowbuf, idxbuf, gbuf, selbuf, sem):
        wid = lax.axis_index("s") * 2 + lax.axis_index("c")
        row0 = wid * ROWS_PER_W
        lanes = lax.iota(jnp.int32, 16)
        zero16 = jnp.zeros((16,), jnp.int32)

        @pl.loop(0, IDXBUF // 16)
        def _(c):
            idxbuf[pl.ds(c * 16, 16)] = zero16

        @pl.loop(0, ROWS_PER_W)
        def _(r):
            row = row0 + r

            @pl.when(row < N_NODES)
            def _():
                pltpu.sync_copy(adj_hbm.at[row], rowbuf)

                def scan_group(grp, off):
                    base = grp * 80
                    ms = [rowbuf[pl.ds(base + q * 16, 16)] > 0
                          for q in range(5)]
                    anyv = ms[0] | ms[1] | ms[2] | ms[3] | ms[4]

                    def hit(off):
                        for q in range(5):
                            mi = ms[q].astype(jnp.int32)
                            s = plsc.cumsum(mi)
                            plsc.store_scatter(
                                idxbuf, [off + (s - mi)],
                                base + q * 16 + lanes, mask=ms[q])
                            off = off + jnp.sum(mi)
                        return off

                    return lax.cond(jnp.any(anyv), hit, lambda o: o, off)

                deg = lax.fori_loop(0, 125, scan_group, jnp.int32(0))
                nsb = jnp.maximum((deg + GCAP - 1) // GCAP, 1)

                def sb_body(sb, _):
                    pltpu.async_copy(
                        f_hbm.at[idxbuf.at[pl.ds(sb * GCAP, GCAP)]],
                        gbuf, sem).wait()
                    rem = jnp.minimum(deg - sb * GCAP, GCAP)

                    for g in range(8):
                        colv = g * 16 + lanes

                        def init_T():
                            return tuple(
                                jnp.where(k < K_TOP - deg,
                                          jnp.float32(0.0), NEG_INF)
                                + jnp.zeros((16,), jnp.float32)
                                for k in range(K_TOP))

                        def load_T():
                            return tuple(
                                selbuf[pl.ds(k * D_FEAT + g * 16, 16)]
                                for k in range(K_TOP))

                        T = lax.cond(sb > 0, load_T, init_T)

                        def j_body(j, T):
                            c = plsc.load_gather(
                                gbuf, [jnp.full((16,), 0, jnp.int32) + j,
                                       colv])
                            Tn = []
                            for k in range(K_TOP):
                                hi = jnp.maximum(T[k], c)
                                c = jnp.minimum(T[k], c)
                                Tn.append(hi)
                            return tuple(Tn)

                        T = lax.fori_loop(0, rem, j_body, T)
                        for k in range(K_TOP):
                            selbuf[pl.ds(k * D_FEAT + g * 16, 16)] = T[k]
                    return 0

                lax.fori_loop(0, nsb, sb_body, 0)
                pltpu.sync_copy(selbuf, out_hbm.at[row])

    return sc_kernel(node_features, adj_matrix)


def _tc_conv(node_features, top8_flat, wnf, wt8, bias2):
    def body(nf_ref, t8_ref, wnf_ref, wt8_ref, b_ref, o_ref):
        acc = jnp.dot(nf_ref[...], wnf_ref[...],
                      preferred_element_type=jnp.float32)
        acc = acc + jnp.dot(t8_ref[...], wt8_ref[...],
                            preferred_element_type=jnp.float32)
        o_ref[...] = acc + b_ref[...]

    return pl.pallas_call(
        body,
        grid=(3, 40),
        in_specs=[
            pl.BlockSpec((256, 128), lambda j, i: (i, 0)),
            pl.BlockSpec((256, 1024), lambda j, i: (i, 0)),
            pl.BlockSpec((128, 1280), lambda j, i: (0, j)),
            pl.BlockSpec((1024, 1280), lambda j, i: (0, j)),
            pl.BlockSpec((1, 1280), lambda j, i: (0, j)),
        ],
        out_specs=pl.BlockSpec((256, 1280), lambda j, i: (i, j)),
        out_shape=jax.ShapeDtypeStruct((N_NODES, 3840), jnp.float32),
        compiler_params=pltpu.CompilerParams(
            dimension_semantics=("arbitrary", "arbitrary")),
    )(node_features, top8_flat, wnf, wt8, bias2)


def kernel(x, node_features, adj_matrix, conv_w, conv_b):
    del x
    top8 = _sc_topk(node_features, adj_matrix)

    h = jnp.arange(120)
    c = jnp.arange(128)
    t = c[:, None] - h[None, :]
    valid = (t >= 0) & (t < 9)
    tc = jnp.clip(t, 0, 8)
    wnf = jnp.where(valid[None], conv_w[:, 0, :][:, tc], 0.0)
    wnf = wnf.transpose(1, 0, 2).reshape(128, 3840)
    wt8 = jnp.where(valid[None, None], conv_w[:, 1:, :][:, :, tc], 0.0)
    wt8 = wt8.transpose(1, 2, 0, 3).reshape(1024, 3840)
    bias2 = jnp.repeat(conv_b, 120).reshape(1, 3840)

    out = _tc_conv(node_features, top8, wnf, wt8, bias2)
    return out.reshape(N_NODES, 32, 120)


# speedup vs baseline: 120.4178x; 6.1807x over previous
import functools

import jax
import jax.numpy as jnp
from jax import lax
from jax.experimental import pallas as pl
from jax.experimental.pallas import tpu as pltpu
from jax.experimental.pallas import tpu_sc as plsc

N_NODES = 10000
D_FEAT = 128
K_TOP = 8
NW = 32
ROWS_PER_W = 313
N_CHUNK = N_NODES // 16
N_CBLK = 40
GCAP = 64
GB = 16
IDXBUF = 10048

NEG_INF = float("-inf")


def _sc_topk(node_features, adj_flat):
    mesh = plsc.VectorSubcoreMesh(core_axis_name="c", subcore_axis_name="s")

    @functools.partial(
        pl.kernel,
        out_type=jax.ShapeDtypeStruct((N_NODES * K_TOP * D_FEAT,),
                                      jnp.float32),
        mesh=mesh,
        scratch_types=[
            pltpu.VMEM((N_NODES,), jnp.int32),
            pltpu.VMEM((N_NODES,), jnp.int32),
            pltpu.VMEM((N_NODES,), jnp.int32),
            pltpu.VMEM((640,), jnp.int32),
            pltpu.VMEM((640,), jnp.int32),
            pltpu.VMEM((16,), jnp.int32),
            pltpu.VMEM((IDXBUF,), jnp.int32),
            pltpu.VMEM((GCAP, D_FEAT), jnp.float32),
            pltpu.VMEM((K_TOP * D_FEAT,), jnp.float32),
            pltpu.SemaphoreType.DMA,
            pltpu.SemaphoreType.DMA,
            pltpu.SemaphoreType.DMA,
        ],
        compiler_params=pltpu.CompilerParams(needs_layout_passes=False),
    )
    def sc_kernel(f_hbm, adj_hbm, out_hbm, rowa, rowb, posbuf, cntbuf,
                  basebuf, scr16, idxbuf, gbuf, selbuf, sema, semb, semg):
        wid = lax.axis_index("s") * 2 + lax.axis_index("c")
        row0 = wid * ROWS_PER_W
        lanes = lax.iota(jnp.int32, 16)
        zero16 = jnp.zeros((16,), jnp.int32)
        m15 = lanes == 15
        full15 = jnp.full((16,), 15, jnp.int32)

        @pl.loop(0, IDXBUF // 16)
        def _(c):
            idxbuf[pl.ds(c * 16, 16)] = zero16

        @pl.loop(0, 640 // 16)
        def _(c):
            cntbuf[pl.ds(c * 16, 16)] = zero16

        def row_dma(local_r, buf, sem):
            pltpu.async_copy(
                adj_hbm.at[pl.ds((row0 + local_r) * N_NODES, N_NODES)],
                buf, sem)

        def row_wait(buf, sem):
            pltpu.make_async_copy(
                adj_hbm.at[pl.ds(0, N_NODES)], buf, sem).wait()

        def prefetch(local_r, buf, sem):
            @pl.when((local_r < ROWS_PER_W) & (row0 + local_r < N_NODES))
            def _():
                row_dma(local_r, buf, sem)

        def process(row, rowbuf):
            @plsc.parallel_loop(0, N_CHUNK, unroll=2)
            def _(c):
                a = rowbuf[pl.ds(c * 16, 16)]
                mi = (a > 0).astype(jnp.int32)
                s = plsc.cumsum(mi)
                posbuf[pl.ds(c * 16, 16)] = s
                plsc.store_scatter(
                    cntbuf, [jnp.full((16,), c, jnp.int32)], s, mask=m15)

            def b_body(t, outer):
                tv = cntbuf[pl.ds(t * 16, 16)]
                cs = plsc.cumsum(tv)
                basebuf[pl.ds(t * 16, 16)] = outer + (cs - tv)
                scr16[...] = cs
                return outer + plsc.load_gather(scr16, [full15])

            outer = lax.fori_loop(0, N_CBLK, b_body, zero16)
            deg = jnp.max(outer)

            @plsc.parallel_loop(0, N_CHUNK, unroll=2)
            def _(c):
                a = rowbuf[pl.ds(c * 16, 16)]
                m = a > 0
                mi = m.astype(jnp.int32)
                s = posbuf[pl.ds(c * 16, 16)]
                base = plsc.load_gather(
                    basebuf, [jnp.full((16,), c, jnp.int32)])
                plsc.store_scatter(
                    idxbuf, [base + (s - mi)], c * 16 + lanes, mask=m)

            nsb = jnp.maximum((deg + GCAP - 1) // GCAP, 1)

            def sb_body(sb, _):
                rem = jnp.minimum(deg - sb * GCAP, GCAP)
                nba = (rem + GB - 1) // GB
                for b in range(GCAP // GB):
                    @pl.when(b < nba)
                    def _():
                        pltpu.async_copy(
                            f_hbm.at[idxbuf.at[
                                pl.ds(sb * GCAP + b * GB, GB)]],
                            gbuf.at[pl.ds(b * GB, GB)], semg)
                for b in range(GCAP // GB):
                    @pl.when(b < nba)
                    def _():
                        pltpu.make_async_copy(
                            f_hbm.at[idxbuf.at[pl.ds(0, GB)]],
                            gbuf.at[pl.ds(0, GB)], semg).wait()

                for g in range(8):
                    colv = g * 16 + lanes

                    def init_T():
                        return tuple(
                            jnp.where(k < K_TOP - deg,
                                      jnp.float32(0.0), NEG_INF)
                            + jnp.zeros((16,), jnp.float32)
                            for k in range(K_TOP))

                    def load_T():
                        return tuple(
                            selbuf[pl.ds(k * D_FEAT + g * 16, 16)]
                            for k in range(K_TOP))

                    T = lax.cond(sb > 0, load_T, init_T)

                    def j_body(j, T):
                        c = plsc.load_gather(
                            gbuf, [jnp.full((16,), 0, jnp.int32) + j, colv])
                        Tn = []
                        for k in range(K_TOP):
                            hi = jnp.maximum(T[k], c)
                            c = jnp.minimum(T[k], c)
                            Tn.append(hi)
                        return tuple(Tn)

                    T = lax.fori_loop(0, rem, j_body, T)
                    for k in range(K_TOP):
                        selbuf[pl.ds(k * D_FEAT + g * 16, 16)] = T[k]
                return 0

            lax.fori_loop(0, nsb, sb_body, 0)
            pltpu.sync_copy(
                selbuf, out_hbm.at[pl.ds(row * (K_TOP * D_FEAT),
                                         K_TOP * D_FEAT)])

        prefetch(0, rowa, sema)

        @pl.loop(0, (ROWS_PER_W + 1) // 2)
        def _(rp):
            r_a = rp * 2
            r_b = rp * 2 + 1

            @pl.when(row0 + r_a < N_NODES)
            def _():
                row_wait(rowa, sema)
                prefetch(r_b, rowb, semb)
                process(row0 + r_a, rowa)

            @pl.when((r_b < ROWS_PER_W) & (row0 + r_b < N_NODES))
            def _():
                row_wait(rowb, semb)
                prefetch(r_b + 1, rowa, sema)
                process(row0 + r_b, rowb)

    return sc_kernel(node_features, adj_flat)


def _tc_conv(node_features, top8_flat, wnf, wt8, bias2):
    def body(nf_ref, t8_ref, wnf_ref, wt8_ref, b_ref, o_ref):
        acc = jnp.dot(nf_ref[...], wnf_ref[...],
                      preferred_element_type=jnp.float32)
        acc = acc + jnp.dot(t8_ref[...], wt8_ref[...],
                            preferred_element_type=jnp.float32)
        o_ref[...] = acc + b_ref[...]

    return pl.pallas_call(
        body,
        grid=(3, 40),
        in_specs=[
            pl.BlockSpec((256, 128), lambda j, i: (i, 0)),
            pl.BlockSpec((256, 1024), lambda j, i: (i, 0)),
            pl.BlockSpec((128, 1280), lambda j, i: (0, j)),
            pl.BlockSpec((1024, 1280), lambda j, i: (0, j)),
            pl.BlockSpec((1, 1280), lambda j, i: (0, j)),
        ],
        out_specs=pl.BlockSpec((256, 1280), lambda j, i: (i, j)),
        out_shape=jax.ShapeDtypeStruct((N_NODES, 3840), jnp.float32),
        compiler_params=pltpu.CompilerParams(
            dimension_semantics=("arbitrary", "arbitrary")),
    )(node_features, top8_flat, wnf, wt8, bias2)


def kernel(x, node_features, adj_matrix, conv_w, conv_b):
    del x
    top8 = _sc_topk(node_features, adj_matrix.reshape(-1))
    top8 = top8.reshape(N_NODES, K_TOP * D_FEAT)

    h = jnp.arange(120)
    c = jnp.arange(128)
    t = c[:, None] - h[None, :]
    valid = (t >= 0) & (t < 9)
    tc = jnp.clip(t, 0, 8)
    wnf = jnp.where(valid[None], conv_w[:, 0, :][:, tc], 0.0)
    wnf = wnf.transpose(1, 0, 2).reshape(128, 3840)
    wt8 = jnp.where(valid[None, None], conv_w[:, 1:, :][:, :, tc], 0.0)
    wt8 = wt8.transpose(1, 2, 0, 3).reshape(1024, 3840)
    bias2 = jnp.repeat(conv_b, 120).reshape(1, 3840)

    out = _tc_conv(node_features, top8, wnf, wt8, bias2)
    return out.reshape(N_NODES, 32, 120)


# speedup vs baseline: 123.8291x; 1.0283x over previous
import functools

import jax
import jax.numpy as jnp
from jax import lax
from jax.experimental import pallas as pl
from jax.experimental.pallas import tpu as pltpu
from jax.experimental.pallas import tpu_sc as plsc

N_NODES = 10000
D_FEAT = 128
K_TOP = 8
NW = 32
ROWS_PER_W = 313
N_CHUNK = N_NODES // 16
N_CBLK = 40
GCAP = 64
GB = 16
IDXBUF = 10048

NEG_INF = float("-inf")


def _sc_topk(node_features, adj_flat):
    mesh = plsc.VectorSubcoreMesh(core_axis_name="c", subcore_axis_name="s")

    @functools.partial(
        pl.kernel,
        out_type=jax.ShapeDtypeStruct((N_NODES * K_TOP * D_FEAT,),
                                      jnp.float32),
        mesh=mesh,
        scratch_types=[
            pltpu.VMEM((N_NODES,), jnp.int32),
            pltpu.VMEM((N_NODES,), jnp.int32),
            pltpu.VMEM((N_NODES,), jnp.int32),
            pltpu.VMEM((640,), jnp.int32),
            pltpu.VMEM((640,), jnp.int32),
            pltpu.VMEM((16,), jnp.int32),
            pltpu.VMEM((IDXBUF,), jnp.int32),
            pltpu.VMEM((GCAP, D_FEAT), jnp.float32),
            pltpu.VMEM((K_TOP * D_FEAT,), jnp.float32),
            pltpu.SemaphoreType.DMA,
            pltpu.SemaphoreType.DMA,
            pltpu.SemaphoreType.DMA,
        ],
        compiler_params=pltpu.CompilerParams(needs_layout_passes=False),
    )
    def sc_kernel(f_hbm, adj_hbm, out_hbm, rowa, rowb, posbuf, cntbuf,
                  basebuf, scr16, idxbuf, gbuf, selbuf, sema, semb, semg):
        wid = lax.axis_index("s") * 2 + lax.axis_index("c")
        row0 = wid * ROWS_PER_W
        lanes = lax.iota(jnp.int32, 16)
        zero16 = jnp.zeros((16,), jnp.int32)
        m15 = lanes == 15
        full15 = jnp.full((16,), 15, jnp.int32)

        @pl.loop(0, IDXBUF // 16)
        def _(c):
            idxbuf[pl.ds(c * 16, 16)] = zero16

        @pl.loop(0, 640 // 16)
        def _(c):
            cntbuf[pl.ds(c * 16, 16)] = zero16

        def row_dma(local_r, buf, sem):
            pltpu.async_copy(
                adj_hbm.at[pl.ds((row0 + local_r) * N_NODES, N_NODES)],
                buf, sem)

        def row_wait(buf, sem):
            pltpu.make_async_copy(
                adj_hbm.at[pl.ds(0, N_NODES)], buf, sem).wait()

        def prefetch(local_r, buf, sem):
            @pl.when((local_r < ROWS_PER_W) & (row0 + local_r < N_NODES))
            def _():
                row_dma(local_r, buf, sem)

        def process(row, rowbuf):
            @plsc.parallel_loop(0, N_CHUNK, unroll=2)
            def _(c):
                a = rowbuf[pl.ds(c * 16, 16)]
                mi = (a > 0).astype(jnp.int32)
                s = plsc.cumsum(mi)
                posbuf[pl.ds(c * 16, 16)] = s
                plsc.store_scatter(
                    cntbuf, [jnp.full((16,), c, jnp.int32)], s, mask=m15)

            def b_body(t, outer):
                tv = cntbuf[pl.ds(t * 16, 16)]
                cs = plsc.cumsum(tv)
                basebuf[pl.ds(t * 16, 16)] = outer + (cs - tv)
                scr16[...] = cs
                return outer + plsc.load_gather(scr16, [full15])

            outer = lax.fori_loop(0, N_CBLK, b_body, zero16)
            deg = jnp.max(outer)

            @plsc.parallel_loop(0, N_CHUNK, unroll=2)
            def _(c):
                a = rowbuf[pl.ds(c * 16, 16)]
                m = a > 0
                mi = m.astype(jnp.int32)
                s = posbuf[pl.ds(c * 16, 16)]
                base = plsc.load_gather(
                    basebuf, [jnp.full((16,), c, jnp.int32)])
                plsc.store_scatter(
                    idxbuf, [base + (s - mi)], c * 16 + lanes, mask=m)

            nsb = jnp.maximum((deg + GCAP - 1) // GCAP, 1)

            def sb_body(sb, _):
                rem = jnp.minimum(deg - sb * GCAP, GCAP)
                nba = (rem + GB - 1) // GB
                for b in range(GCAP // GB):
                    @pl.when(b < nba)
                    def _():
                        pltpu.async_copy(
                            f_hbm.at[idxbuf.at[
                                pl.ds(sb * GCAP + b * GB, GB)]],
                            gbuf.at[pl.ds(b * GB, GB)], semg)
                for b in range(GCAP // GB):
                    @pl.when(b < nba)
                    def _():
                        pltpu.make_async_copy(
                            f_hbm.at[idxbuf.at[pl.ds(0, GB)]],
                            gbuf.at[pl.ds(0, GB)], semg).wait()

                for g in range(8):
                    colv = g * 16 + lanes

                    def init_T():
                        return tuple(
                            jnp.where(k < K_TOP - deg,
                                      jnp.float32(0.0), NEG_INF)
                            + jnp.zeros((16,), jnp.float32)
                            for k in range(K_TOP))

                    def load_T():
                        return tuple(
                            selbuf[pl.ds(k * D_FEAT + g * 16, 16)]
                            for k in range(K_TOP))

                    T = lax.cond(sb > 0, load_T, init_T)

                    def j_body(j, T):
                        c = plsc.load_gather(
                            gbuf, [jnp.full((16,), 0, jnp.int32) + j, colv])
                        Tn = []
                        for k in range(K_TOP):
                            hi = jnp.maximum(T[k], c)
                            c = jnp.minimum(T[k], c)
                            Tn.append(hi)
                        return tuple(Tn)

                    T = lax.fori_loop(0, rem, j_body, T)
                    for k in range(K_TOP):
                        selbuf[pl.ds(k * D_FEAT + g * 16, 16)] = T[k]
                return 0

            lax.fori_loop(0, nsb, sb_body, 0)
            pltpu.sync_copy(
                selbuf, out_hbm.at[pl.ds(row * (K_TOP * D_FEAT),
                                         K_TOP * D_FEAT)])

        prefetch(0, rowa, sema)

        @pl.loop(0, (ROWS_PER_W + 1) // 2)
        def _(rp):
            r_a = rp * 2
            r_b = rp * 2 + 1

            @pl.when(row0 + r_a < N_NODES)
            def _():
                row_wait(rowa, sema)
                prefetch(r_b, rowb, semb)
                process(row0 + r_a, rowa)

            @pl.when((r_b < ROWS_PER_W) & (row0 + r_b < N_NODES))
            def _():
                row_wait(rowb, semb)
                prefetch(r_b + 1, rowa, sema)
                process(row0 + r_b, rowb)

    return sc_kernel(node_features, adj_flat)


def _tc_conv(node_features, top8_flat, wnf, wt8, bias2):
    def body(nf_ref, t8_ref, wnf_ref, wt8_ref, b_ref, o_ref):
        acc = jnp.dot(nf_ref[...], wnf_ref[...],
                      preferred_element_type=jnp.float32)
        acc = acc + jnp.dot(t8_ref[...], wt8_ref[...],
                            preferred_element_type=jnp.float32)
        o_ref[...] = acc + b_ref[...]

    return pl.pallas_call(
        body,
        grid=(3, 40),
        in_specs=[
            pl.BlockSpec((256, 128), lambda j, i: (i, 0)),
            pl.BlockSpec((256, 1024), lambda j, i: (i, 0)),
            pl.BlockSpec((128, 1280), lambda j, i: (0, j)),
            pl.BlockSpec((1024, 1280), lambda j, i: (0, j)),
            pl.BlockSpec((1, 1280), lambda j, i: (0, j)),
        ],
        out_specs=pl.BlockSpec((256, 1280), lambda j, i: (i, j)),
        out_shape=jax.ShapeDtypeStruct((N_NODES, 3840), jnp.float32),
        compiler_params=pltpu.CompilerParams(
            dimension_semantics=("arbitrary", "arbitrary")),
    )(node_features.astype(jnp.bfloat16), top8_flat.astype(jnp.bfloat16),
      wnf.astype(jnp.bfloat16), wt8.astype(jnp.bfloat16), bias2)


def kernel(x, node_features, adj_matrix, conv_w, conv_b):
    del x
    top8 = _sc_topk(node_features, adj_matrix.reshape(-1))
    top8 = top8.reshape(N_NODES, K_TOP * D_FEAT)

    h = jnp.arange(120)
    c = jnp.arange(128)
    t = c[:, None] - h[None, :]
    valid = (t >= 0) & (t < 9)
    tc = jnp.clip(t, 0, 8)
    wnf = jnp.where(valid[None], conv_w[:, 0, :][:, tc], 0.0)
    wnf = wnf.transpose(1, 0, 2).reshape(128, 3840)
    wt8 = jnp.where(valid[None, None], conv_w[:, 1:, :][:, :, tc], 0.0)
    wt8 = wt8.transpose(1, 2, 0, 3).reshape(1024, 3840)
    bias2 = jnp.repeat(conv_b, 120).reshape(1, 3840)

    out = _tc_conv(node_features, top8, wnf, wt8, bias2)
    return out.reshape(N_NODES, 32, 120)


# speedup vs baseline: 129.8195x; 1.0484x over previous
import functools

import jax
import jax.numpy as jnp
from jax import lax
from jax.experimental import pallas as pl
from jax.experimental.pallas import tpu as pltpu
from jax.experimental.pallas import tpu_sc as plsc

N_NODES = 10000
D_FEAT = 128
K_TOP = 8
NW = 32
ROWS_PER_W = 313
N_CHUNK = N_NODES // 16
N_CBLK = 40
GCAP = 64
GB = 16
IDXBUF = 10048

NEG_INF = float("-inf")


def _sc_topk(node_features, adj_flat):
    mesh = plsc.VectorSubcoreMesh(core_axis_name="c", subcore_axis_name="s")

    @functools.partial(
        pl.kernel,
        out_type=jax.ShapeDtypeStruct((N_NODES * K_TOP * D_FEAT,),
                                      jnp.float32),
        mesh=mesh,
        scratch_types=[
            pltpu.VMEM((N_NODES,), jnp.int32),
            pltpu.VMEM((N_NODES,), jnp.int32),
            pltpu.VMEM((N_NODES,), jnp.int32),
            pltpu.VMEM((640,), jnp.int32),
            pltpu.VMEM((640,), jnp.int32),
            pltpu.VMEM((16,), jnp.int32),
            pltpu.VMEM((IDXBUF,), jnp.int32),
            pltpu.VMEM((GCAP, D_FEAT), jnp.float32),
            pltpu.VMEM((K_TOP * D_FEAT,), jnp.float32),
            pltpu.SemaphoreType.DMA,
            pltpu.SemaphoreType.DMA,
            pltpu.SemaphoreType.DMA,
        ],
        compiler_params=pltpu.CompilerParams(needs_layout_passes=False),
    )
    def sc_kernel(f_hbm, adj_hbm, out_hbm, rowa, rowb, posbuf, cntbuf,
                  basebuf, scr16, idxbuf, gbuf, selbuf, sema, semb, semg):
        wid = lax.axis_index("s") * 2 + lax.axis_index("c")
        row0 = wid * ROWS_PER_W
        lanes = lax.iota(jnp.int32, 16)
        zero16 = jnp.zeros((16,), jnp.int32)
        m15 = lanes == 15
        full15 = jnp.full((16,), 15, jnp.int32)

        @pl.loop(0, IDXBUF // 16)
        def _(c):
            idxbuf[pl.ds(c * 16, 16)] = zero16

        @pl.loop(0, 640 // 16)
        def _(c):
            cntbuf[pl.ds(c * 16, 16)] = zero16

        def row_dma(local_r, buf, sem):
            pltpu.async_copy(
                adj_hbm.at[pl.ds((row0 + local_r) * N_NODES, N_NODES)],
                buf, sem)

        def row_wait(buf, sem):
            pltpu.make_async_copy(
                adj_hbm.at[pl.ds(0, N_NODES)], buf, sem).wait()

        def prefetch(local_r, buf, sem):
            @pl.when((local_r < ROWS_PER_W) & (row0 + local_r < N_NODES))
            def _():
                row_dma(local_r, buf, sem)

        def process(row, rowbuf):
            @plsc.parallel_loop(0, N_CHUNK, unroll=2)
            def _(c):
                a = rowbuf[pl.ds(c * 16, 16)]
                mi = (a > 0).astype(jnp.int32)
                s = plsc.cumsum(mi)
                posbuf[pl.ds(c * 16, 16)] = s
                plsc.store_scatter(
                    cntbuf, [jnp.full((16,), c, jnp.int32)], s, mask=m15)

            def b_body(t, outer):
                tv = cntbuf[pl.ds(t * 16, 16)]
                cs = plsc.cumsum(tv)
                basebuf[pl.ds(t * 16, 16)] = outer + (cs - tv)
                scr16[...] = cs
                return outer + plsc.load_gather(scr16, [full15])

            outer = lax.fori_loop(0, N_CBLK, b_body, zero16)
            deg = jnp.max(outer)

            @plsc.parallel_loop(0, N_CHUNK, unroll=2)
            def _(c):
                a = rowbuf[pl.ds(c * 16, 16)]
                m = a > 0
                mi = m.astype(jnp.int32)
                s = posbuf[pl.ds(c * 16, 16)]
                base = plsc.load_gather(
                    basebuf, [jnp.full((16,), c, jnp.int32)])
                plsc.store_scatter(
                    idxbuf, [base + (s - mi)], c * 16 + lanes, mask=m)

            nsb = jnp.maximum((deg + GCAP - 1) // GCAP, 1)

            def sb_body(sb, _):
                rem = jnp.minimum(deg - sb * GCAP, GCAP)
                nba = (rem + GB - 1) // GB
                for b in range(GCAP // GB):
                    @pl.when(b < nba)
                    def _():
                        pltpu.async_copy(
                            f_hbm.at[idxbuf.at[
                                pl.ds(sb * GCAP + b * GB, GB)]],
                            gbuf.at[pl.ds(b * GB, GB)], semg)
                for b in range(GCAP // GB):
                    @pl.when(b < nba)
                    def _():
                        pltpu.make_async_copy(
                            f_hbm.at[idxbuf.at[pl.ds(0, GB)]],
                            gbuf.at[pl.ds(0, GB)], semg).wait()

                for g in range(8):
                    colv = g * 16 + lanes

                    def init_T():
                        return tuple(
                            jnp.where(k < K_TOP - deg,
                                      jnp.float32(0.0), NEG_INF)
                            + jnp.zeros((16,), jnp.float32)
                            for k in range(K_TOP))

                    def load_T():
                        return tuple(
                            selbuf[pl.ds(k * D_FEAT + g * 16, 16)]
                            for k in range(K_TOP))

                    T = lax.cond(sb > 0, load_T, init_T)

                    def j_body(j, T):
                        c = plsc.load_gather(
                            gbuf, [jnp.full((16,), 0, jnp.int32) + j, colv])
                        Tn = []
                        for k in range(K_TOP):
                            hi = jnp.maximum(T[k], c)
                            c = jnp.minimum(T[k], c)
                            Tn.append(hi)
                        return tuple(Tn)

                    T = lax.fori_loop(0, rem, j_body, T)
                    for k in range(K_TOP):
                        selbuf[pl.ds(k * D_FEAT + g * 16, 16)] = T[k]
                return 0

            lax.fori_loop(0, nsb, sb_body, 0)
            pltpu.sync_copy(
                selbuf, out_hbm.at[pl.ds(row * (K_TOP * D_FEAT),
                                         K_TOP * D_FEAT)])

        prefetch(0, rowa, sema)

        @pl.loop(0, (ROWS_PER_W + 1) // 2)
        def _(rp):
            r_a = rp * 2
            r_b = rp * 2 + 1

            @pl.when(row0 + r_a < N_NODES)
            def _():
                row_wait(rowa, sema)
                prefetch(r_b, rowb, semb)
                process(row0 + r_a, rowa)

            @pl.when((r_b < ROWS_PER_W) & (row0 + r_b < N_NODES))
            def _():
                row_wait(rowb, semb)
                prefetch(r_b + 1, rowa, sema)
                process(row0 + r_b, rowb)

    return sc_kernel(node_features, adj_flat)


def _tc_conv(node_features, top8_flat, wnf, wt8, bias2):
    def body(nf_ref, t8_ref, wnf_ref, wt8_ref, b_ref, o_ref):
        acc = jnp.dot(nf_ref[...], wnf_ref[...],
                      preferred_element_type=jnp.float32)
        acc = acc + jnp.dot(t8_ref[...], wt8_ref[...],
                            preferred_element_type=jnp.float32)
        o_ref[...] = acc + b_ref[...]

    return pl.pallas_call(
        body,
        grid=(3, 40),
        in_specs=[
            pl.BlockSpec((256, 128), lambda j, i: (i, 0)),
            pl.BlockSpec((256, 1024), lambda j, i: (i, 0)),
            pl.BlockSpec((128, 1280), lambda j, i: (0, j)),
            pl.BlockSpec((1024, 1280), lambda j, i: (0, j)),
            pl.BlockSpec((1, 1280), lambda j, i: (0, j)),
        ],
        out_specs=pl.BlockSpec((256, 1280), lambda j, i: (i, j)),
        out_shape=jax.ShapeDtypeStruct((N_NODES, 3840), jnp.float32),
        compiler_params=pltpu.CompilerParams(
            dimension_semantics=("arbitrary", "arbitrary")),
    )(node_features.astype(jnp.bfloat16), top8_flat.astype(jnp.bfloat16),
      wnf.astype(jnp.bfloat16), wt8.astype(jnp.bfloat16), bias2)


def kernel(x, node_features, adj_matrix, conv_w, conv_b):
    del x
    top8 = _sc_topk(node_features, adj_matrix.reshape(-1))
    top8 = top8.reshape(N_NODES, K_TOP * D_FEAT)

    pw = jnp.pad(conv_w, ((0, 0), (0, 0), (0, 120)))
    band = jnp.tile(pw, (1, 1, 120))[:, :, :120 * 128]
    band = band.reshape(32, 9, 120, 128)
    wnf = band[:, 0].transpose(2, 0, 1).reshape(128, 3840)
    wt8 = band[:, 1:].transpose(1, 3, 0, 2).reshape(1024, 3840)
    bias2 = jnp.repeat(conv_b, 120).reshape(1, 3840)

    out = _tc_conv(node_features, top8, wnf, wt8, bias2)
    return out.reshape(N_NODES, 32, 120)


# speedup vs baseline: 153.8058x; 1.1848x over previous
import functools

import jax
import jax.numpy as jnp
from jax import lax
from jax.experimental import pallas as pl
from jax.experimental.pallas import tpu as pltpu
from jax.experimental.pallas import tpu_sc as plsc

N_NODES = 10000
D_FEAT = 128
K_TOP = 8
NW = 32
ROWS_PER_W = 313
N_CHUNK = N_NODES // 16
N_CBLK = 40
GCAP = 64
GB = 16
IDXBUF = 10048

NEG_INF = float("-inf")


def _sc_topk(node_features, adj_matrix):
    mesh = plsc.VectorSubcoreMesh(core_axis_name="c", subcore_axis_name="s")

    @functools.partial(
        pl.kernel,
        out_type=jax.ShapeDtypeStruct((N_NODES * K_TOP * D_FEAT,),
                                      jnp.float32),
        mesh=mesh,
        scratch_types=[
            pltpu.VMEM((N_NODES,), jnp.int32),
            pltpu.VMEM((N_NODES,), jnp.int32),
            pltpu.VMEM((N_NODES,), jnp.int32),
            pltpu.VMEM((640,), jnp.int32),
            pltpu.VMEM((640,), jnp.int32),
            pltpu.VMEM((16,), jnp.int32),
            pltpu.VMEM((IDXBUF,), jnp.int32),
            pltpu.VMEM((GCAP, D_FEAT), jnp.float32),
            pltpu.VMEM((K_TOP * D_FEAT,), jnp.float32),
            pltpu.SemaphoreType.DMA,
            pltpu.SemaphoreType.DMA,
            pltpu.SemaphoreType.DMA,
        ],
        compiler_params=pltpu.CompilerParams(needs_layout_passes=False),
    )
    def sc_kernel(f_hbm, adj_hbm, out_hbm, rowa, rowb, posbuf, cntbuf,
                  basebuf, scr16, idxbuf, gbuf, selbuf, sema, semb, semg):
        wid = lax.axis_index("s") * 2 + lax.axis_index("c")
        row0 = wid * ROWS_PER_W
        lanes = lax.iota(jnp.int32, 16)
        zero16 = jnp.zeros((16,), jnp.int32)
        m15 = lanes == 15
        full15 = jnp.full((16,), 15, jnp.int32)

        @pl.loop(0, IDXBUF // 16)
        def _(c):
            idxbuf[pl.ds(c * 16, 16)] = zero16

        @pl.loop(0, 640 // 16)
        def _(c):
            cntbuf[pl.ds(c * 16, 16)] = zero16

        def row_dma(local_r, buf, sem):
            pltpu.async_copy(adj_hbm.at[row0 + local_r], buf, sem)

        def row_wait(buf, sem):
            pltpu.make_async_copy(adj_hbm.at[0], buf, sem).wait()

        def prefetch(local_r, buf, sem):
            @pl.when((local_r < ROWS_PER_W) & (row0 + local_r < N_NODES))
            def _():
                row_dma(local_r, buf, sem)

        def process(row, rowbuf):
            @plsc.parallel_loop(0, N_CHUNK, unroll=2)
            def _(c):
                a = rowbuf[pl.ds(c * 16, 16)]
                mi = (a > 0).astype(jnp.int32)
                s = plsc.cumsum(mi)
                posbuf[pl.ds(c * 16, 16)] = s
                plsc.store_scatter(
                    cntbuf, [jnp.full((16,), c, jnp.int32)], s, mask=m15)

            def b_body(t, outer):
                tv = cntbuf[pl.ds(t * 16, 16)]
                cs = plsc.cumsum(tv)
                basebuf[pl.ds(t * 16, 16)] = outer + (cs - tv)
                scr16[...] = cs
                return outer + plsc.load_gather(scr16, [full15])

            outer = lax.fori_loop(0, N_CBLK, b_body, zero16)
            deg = jnp.max(outer)

            @plsc.parallel_loop(0, N_CHUNK, unroll=2)
            def _(c):
                a = rowbuf[pl.ds(c * 16, 16)]
                m = a > 0
                mi = m.astype(jnp.int32)
                s = posbuf[pl.ds(c * 16, 16)]
                base = plsc.load_gather(
                    basebuf, [jnp.full((16,), c, jnp.int32)])
                plsc.store_scatter(
                    idxbuf, [base + (s - mi)], c * 16 + lanes, mask=m)

            nsb = jnp.maximum((deg + GCAP - 1) // GCAP, 1)

            def sb_body(sb, _):
                rem = jnp.minimum(deg - sb * GCAP, GCAP)
                nba = (rem + GB - 1) // GB
                for b in range(GCAP // GB):
                    @pl.when(b < nba)
                    def _():
                        pltpu.async_copy(
                            f_hbm.at[idxbuf.at[
                                pl.ds(sb * GCAP + b * GB, GB)]],
                            gbuf.at[pl.ds(b * GB, GB)], semg)
                for b in range(GCAP // GB):
                    @pl.when(b < nba)
                    def _():
                        pltpu.make_async_copy(
                            f_hbm.at[idxbuf.at[pl.ds(0, GB)]],
                            gbuf.at[pl.ds(0, GB)], semg).wait()

                for g in range(8):
                    colv = g * 16 + lanes

                    def init_T():
                        return tuple(
                            jnp.where(k < K_TOP - deg,
                                      jnp.float32(0.0), NEG_INF)
                            + jnp.zeros((16,), jnp.float32)
                            for k in range(K_TOP))

                    def load_T():
                        return tuple(
                            selbuf[pl.ds(k * D_FEAT + g * 16, 16)]
                            for k in range(K_TOP))

                    T = lax.cond(sb > 0, load_T, init_T)

                    def j_body(j, T):
                        c = plsc.load_gather(
                            gbuf, [jnp.full((16,), 0, jnp.int32) + j, colv])
                        Tn = []
                        for k in range(K_TOP):
                            hi = jnp.maximum(T[k], c)
                            c = jnp.minimum(T[k], c)
                            Tn.append(hi)
                        return tuple(Tn)

                    T = lax.fori_loop(0, rem, j_body, T)
                    for k in range(K_TOP):
                        selbuf[pl.ds(k * D_FEAT + g * 16, 16)] = T[k]
                return 0

            lax.fori_loop(0, nsb, sb_body, 0)
            pltpu.sync_copy(
                selbuf, out_hbm.at[pl.ds(row * (K_TOP * D_FEAT),
                                         K_TOP * D_FEAT)])

        prefetch(0, rowa, sema)

        @pl.loop(0, (ROWS_PER_W + 1) // 2)
        def _(rp):
            r_a = rp * 2
            r_b = rp * 2 + 1

            @pl.when(row0 + r_a < N_NODES)
            def _():
                row_wait(rowa, sema)
                prefetch(r_b, rowb, semb)
                process(row0 + r_a, rowa)

            @pl.when((r_b < ROWS_PER_W) & (row0 + r_b < N_NODES))
            def _():
                row_wait(rowb, semb)
                prefetch(r_b + 1, rowa, sema)
                process(row0 + r_b, rowb)

    return sc_kernel(node_features, adj_matrix)


def _tc_conv(node_features, top8_flat, wnf, wt8, bias2):
    def body(nf_ref, t8_ref, wnf_ref, wt8_ref, b_ref, o_ref):
        acc = jnp.dot(nf_ref[...], wnf_ref[...],
                      preferred_element_type=jnp.float32)
        acc = acc + jnp.dot(t8_ref[...], wt8_ref[...],
                            preferred_element_type=jnp.float32)
        o_ref[...] = acc + b_ref[...]

    return pl.pallas_call(
        body,
        grid=(3, 40),
        in_specs=[
            pl.BlockSpec((256, 128), lambda j, i: (i, 0)),
            pl.BlockSpec((256, 1024), lambda j, i: (i, 0)),
            pl.BlockSpec((128, 1280), lambda j, i: (0, j)),
            pl.BlockSpec((1024, 1280), lambda j, i: (0, j)),
            pl.BlockSpec((1, 1280), lambda j, i: (0, j)),
        ],
        out_specs=pl.BlockSpec((256, 1280), lambda j, i: (i, j)),
        out_shape=jax.ShapeDtypeStruct((N_NODES, 3840), jnp.float32),
        compiler_params=pltpu.CompilerParams(
            dimension_semantics=("arbitrary", "arbitrary")),
    )(node_features.astype(jnp.bfloat16), top8_flat.astype(jnp.bfloat16),
      wnf.astype(jnp.bfloat16), wt8.astype(jnp.bfloat16), bias2)


def kernel(x, node_features, adj_matrix, conv_w, conv_b):
    del x
    top8 = _sc_topk(node_features, adj_matrix)
    top8 = top8.reshape(N_NODES, K_TOP * D_FEAT)

    pw = jnp.pad(conv_w, ((0, 0), (0, 0), (0, 120)))
    band = jnp.tile(pw, (1, 1, 120))[:, :, :120 * 128]
    band = band.reshape(32, 9, 120, 128)
    wnf = band[:, 0].transpose(2, 0, 1).reshape(128, 3840)
    wt8 = band[:, 1:].transpose(1, 3, 0, 2).reshape(1024, 3840)
    bias2 = jnp.repeat(conv_b, 120).reshape(1, 3840)

    out = _tc_conv(node_features, top8, wnf, wt8, bias2)
    return out.reshape(N_NODES, 32, 120)


# speedup vs baseline: 163.9530x; 1.0660x over previous
import functools

import jax
import jax.numpy as jnp
from jax import lax
from jax.experimental import pallas as pl
from jax.experimental.pallas import tpu as pltpu
from jax.experimental.pallas import tpu_sc as plsc

N_NODES = 10000
D_FEAT = 128
K_TOP = 8
NW = 32
ROWS_PER_W = 313
N_CHUNK = N_NODES // 16
N_CBLK = 40
GCAP = 64
GB = 16
IDXBUF = 10048

NEG_INF = float("-inf")


def _sc_topk(node_features, adj_matrix):
    mesh = plsc.VectorSubcoreMesh(core_axis_name="c", subcore_axis_name="s")

    @functools.partial(
        pl.kernel,
        out_type=jax.ShapeDtypeStruct((N_NODES * K_TOP * D_FEAT,),
                                      jnp.float32),
        mesh=mesh,
        scratch_types=[
            pltpu.VMEM((N_NODES,), jnp.int32),
            pltpu.VMEM((N_NODES,), jnp.int32),
            pltpu.VMEM((N_NODES,), jnp.int32),
            pltpu.VMEM((640,), jnp.int32),
            pltpu.VMEM((640,), jnp.int32),
            pltpu.VMEM((16,), jnp.int32),
            pltpu.VMEM((IDXBUF,), jnp.int32),
            pltpu.VMEM((GCAP, D_FEAT), jnp.float32),
            pltpu.VMEM((K_TOP * D_FEAT,), jnp.float32),
            pltpu.SemaphoreType.DMA,
            pltpu.SemaphoreType.DMA,
            pltpu.SemaphoreType.DMA,
        ],
        compiler_params=pltpu.CompilerParams(needs_layout_passes=False),
    )
    def sc_kernel(f_hbm, adj_hbm, out_hbm, rowa, rowb, posbuf, cntbuf,
                  basebuf, scr16, idxbuf, gbuf, selbuf, sema, semb, semg):
        wid = lax.axis_index("s") * 2 + lax.axis_index("c")
        row0 = wid * ROWS_PER_W
        lanes = lax.iota(jnp.int32, 16)
        zero16 = jnp.zeros((16,), jnp.int32)
        m15 = lanes == 15
        full15 = jnp.full((16,), 15, jnp.int32)

        @pl.loop(0, IDXBUF // 16)
        def _(c):
            idxbuf[pl.ds(c * 16, 16)] = zero16

        @pl.loop(0, 640 // 16)
        def _(c):
            cntbuf[pl.ds(c * 16, 16)] = zero16

        def row_dma(local_r, buf, sem):
            pltpu.async_copy(adj_hbm.at[row0 + local_r], buf, sem)

        def row_wait(buf, sem):
            pltpu.make_async_copy(adj_hbm.at[0], buf, sem).wait()

        def prefetch(local_r, buf, sem):
            @pl.when((local_r < ROWS_PER_W) & (row0 + local_r < N_NODES))
            def _():
                row_dma(local_r, buf, sem)

        def process(row, rowbuf):
            @plsc.parallel_loop(0, N_CHUNK, unroll=5)
            def _(c):
                a = rowbuf[pl.ds(c * 16, 16)]
                mi = (a > 0).astype(jnp.int32)
                s = plsc.cumsum(mi)
                posbuf[pl.ds(c * 16, 16)] = s
                plsc.store_scatter(
                    cntbuf, [jnp.full((16,), c, jnp.int32)], s, mask=m15)

            def b_body(t, outer):
                tv = cntbuf[pl.ds(t * 16, 16)]
                cs = plsc.cumsum(tv)
                basebuf[pl.ds(t * 16, 16)] = outer + (cs - tv)
                scr16[...] = cs
                return outer + plsc.load_gather(scr16, [full15])

            outer = lax.fori_loop(0, N_CBLK, b_body, zero16)
            deg = jnp.max(outer)

            @plsc.parallel_loop(0, N_CHUNK, unroll=5)
            def _(c):
                a = rowbuf[pl.ds(c * 16, 16)]
                m = a > 0
                mi = m.astype(jnp.int32)
                s = posbuf[pl.ds(c * 16, 16)]
                base = plsc.load_gather(
                    basebuf, [jnp.full((16,), c, jnp.int32)])
                plsc.store_scatter(
                    idxbuf, [base + (s - mi)], c * 16 + lanes, mask=m)

            nsb = jnp.maximum((deg + GCAP - 1) // GCAP, 1)

            def sb_body(sb, _):
                rem = jnp.minimum(deg - sb * GCAP, GCAP)
                nba = (rem + GB - 1) // GB
                for b in range(GCAP // GB):
                    @pl.when(b < nba)
                    def _():
                        pltpu.async_copy(
                            f_hbm.at[idxbuf.at[
                                pl.ds(sb * GCAP + b * GB, GB)]],
                            gbuf.at[pl.ds(b * GB, GB)], semg)
                for b in range(GCAP // GB):
                    @pl.when(b < nba)
                    def _():
                        pltpu.make_async_copy(
                            f_hbm.at[idxbuf.at[pl.ds(0, GB)]],
                            gbuf.at[pl.ds(0, GB)], semg).wait()

                for g in range(8):
                    colv = g * 16 + lanes

                    def init_T():
                        return tuple(
                            jnp.where(k < K_TOP - deg,
                                      jnp.float32(0.0), NEG_INF)
                            + jnp.zeros((16,), jnp.float32)
                            for k in range(K_TOP))

                    def load_T():
                        return tuple(
                            selbuf[pl.ds(k * D_FEAT + g * 16, 16)]
                            for k in range(K_TOP))

                    T = lax.cond(sb > 0, load_T, init_T)

                    def j_body(j, T):
                        c = plsc.load_gather(
                            gbuf, [jnp.full((16,), 0, jnp.int32) + j, colv])
                        Tn = []
                        for k in range(K_TOP):
                            hi = jnp.maximum(T[k], c)
                            c = jnp.minimum(T[k], c)
                            Tn.append(hi)
                        return tuple(Tn)

                    T = lax.fori_loop(0, rem, j_body, T)
                    for k in range(K_TOP):
                        selbuf[pl.ds(k * D_FEAT + g * 16, 16)] = T[k]
                return 0

            lax.fori_loop(0, nsb, sb_body, 0)
            pltpu.sync_copy(
                selbuf, out_hbm.at[pl.ds(row * (K_TOP * D_FEAT),
                                         K_TOP * D_FEAT)])

        prefetch(0, rowa, sema)

        @pl.loop(0, (ROWS_PER_W + 1) // 2)
        def _(rp):
            r_a = rp * 2
            r_b = rp * 2 + 1

            @pl.when(row0 + r_a < N_NODES)
            def _():
                row_wait(rowa, sema)
                prefetch(r_b, rowb, semb)
                process(row0 + r_a, rowa)

            @pl.when((r_b < ROWS_PER_W) & (row0 + r_b < N_NODES))
            def _():
                row_wait(rowb, semb)
                prefetch(r_b + 1, rowa, sema)
                process(row0 + r_b, rowb)

    return sc_kernel(node_features, adj_matrix)


def _tc_conv(node_features, top8_flat, wnf, wt8, bias2):
    def body(nf_ref, t8_ref, wnf_ref, wt8_ref, b_ref, o_ref):
        acc = jnp.dot(nf_ref[...], wnf_ref[...],
                      preferred_element_type=jnp.float32)
        acc = acc + jnp.dot(t8_ref[...], wt8_ref[...],
                            preferred_element_type=jnp.float32)
        o_ref[...] = acc + b_ref[...]

    return pl.pallas_call(
        body,
        grid=(3, 40),
        in_specs=[
            pl.BlockSpec((256, 128), lambda j, i: (i, 0)),
            pl.BlockSpec((256, 1024), lambda j, i: (i, 0)),
            pl.BlockSpec((128, 1280), lambda j, i: (0, j)),
            pl.BlockSpec((1024, 1280), lambda j, i: (0, j)),
            pl.BlockSpec((1, 1280), lambda j, i: (0, j)),
        ],
        out_specs=pl.BlockSpec((256, 1280), lambda j, i: (i, j)),
        out_shape=jax.ShapeDtypeStruct((N_NODES, 3840), jnp.float32),
        compiler_params=pltpu.CompilerParams(
            dimension_semantics=("arbitrary", "arbitrary")),
    )(node_features.astype(jnp.bfloat16), top8_flat.astype(jnp.bfloat16),
      wnf.astype(jnp.bfloat16), wt8.astype(jnp.bfloat16), bias2)


def kernel(x, node_features, adj_matrix, conv_w, conv_b):
    del x
    top8 = _sc_topk(node_features, adj_matrix)
    top8 = top8.reshape(N_NODES, K_TOP * D_FEAT)

    pw = jnp.pad(conv_w, ((0, 0), (0, 0), (0, 120)))
    band = jnp.tile(pw, (1, 1, 120))[:, :, :120 * 128]
    band = band.reshape(32, 9, 120, 128)
    wnf = band[:, 0].transpose(2, 0, 1).reshape(128, 3840)
    wt8 = band[:, 1:].transpose(1, 3, 0, 2).reshape(1024, 3840)
    bias2 = jnp.repeat(conv_b, 120).reshape(1, 3840)

    out = _tc_conv(node_features, top8, wnf, wt8, bias2)
    return out.reshape(N_NODES, 32, 120)
